# baseline (speedup 1.0000x reference)
import functools

import jax
import jax.numpy as jnp
from jax import lax
from jax.experimental import pallas as pl
from jax.experimental.pallas import tpu as pltpu
from jax.experimental.pallas import tpu_sc as plsc

B, S, D, E, F, K = 1, 2048, 768, 8, 2048, 2
D2 = D // 2
N = S * K
T = 128
P = N + E * T
NT = P // T
NW = 32
PPW = N // NW
SPW = P // NW
TPW = S // NW

_mesh = functools.partial(
    plsc.VectorSubcoreMesh, core_axis_name="c", subcore_axis_name="s",
    num_cores=2, num_subcores=16)


def _wid():
    return lax.axis_index("s") * 2 + lax.axis_index("c")


def _gate_body(x_ref, gw_ref, gb_ref, e_ref, w_ref):
    x = x_ref[...]
    logits = jnp.dot(x, gw_ref[...], preferred_element_type=jnp.float32)
    logits = logits + gb_ref[...]
    iota = lax.broadcasted_iota(jnp.int32, (S, E), 1)
    m0 = jnp.max(logits, axis=1, keepdims=True)
    e0 = jnp.min(jnp.where(logits == m0, iota, E), axis=1, keepdims=True)
    mask0 = iota == e0
    l1m = jnp.where(mask0, jnp.float32(-1e30), logits)
    m1 = jnp.max(l1m, axis=1, keepdims=True)
    e1 = jnp.min(jnp.where(l1m == m1, iota, E), axis=1, keepdims=True)
    a = jnp.exp(m1 - m0)
    w0 = 1.0 / (1.0 + a)
    e_ref[...] = jnp.concatenate([e0, e1], axis=1)
    w_ref[...] = jnp.concatenate([w0, 1.0 - w0], axis=1)


def _gating(x2d, gate_w, gate_b):
    return pl.pallas_call(
        _gate_body,
        out_shape=(
            jax.ShapeDtypeStruct((S, K), jnp.int32),
            jax.ShapeDtypeStruct((S, K), jnp.float32),
        ),
    )(x2d, gate_w, gate_b.reshape(1, E))


def _lane(vec, e):
    iota16 = lax.iota(jnp.int32, 16)
    return jnp.max(jnp.where(iota16 == e, vec, jnp.int32(-2147483647)))


def _route_body(eflat, wflat, x2d, xs, wsort, invpos, gtile, nact,
                eb, wb, sbuf, shv, posa, posb, toka, tokb,
                rowsa, rowsb, s16, sh, sem, sem2, sem3):
    sid = lax.axis_index("s")
    cid = lax.axis_index("c")
    wid = sid * 2 + cid
    iota16 = lax.iota(jnp.int32, 16)
    blk = N // 16
    pltpu.sync_copy(eflat.at[pl.ds(sid * blk, blk)], eb)
    pltpu.sync_copy(wflat.at[pl.ds(wid * PPW, PPW)], wb)

    loc = jnp.zeros((16,), jnp.int32)
    hlf = jnp.zeros((16,), jnp.int32)
    for cc in range(blk // 16):
        if cc == blk // 32:
            hlf = loc
        ev = eb[pl.ds(cc * 16, 16)]
        upd = jnp.zeros((16,), jnp.int32)
        for e in range(E):
            pc = plsc.all_reduce_population_count(ev == e)
            upd = upd + jnp.where(iota16 == e, pc, 0)
        loc = loc + upd
    sbuf[pl.ds(0, 16)] = loc
    sbuf[pl.ds(16, 16)] = hlf
    pltpu.sync_copy(sbuf, sh.at[pl.ds(sid * 32, 32)])
    plsc.subcore_barrier()
    pltpu.sync_copy(sh, shv)

    cnt = jnp.zeros((16,), jnp.int32)
    pre = jnp.zeros((16,), jnp.int32)
    for s2 in range(16):
        row = shv[pl.ds(s2 * 32, 16)]
        cnt = cnt + row
        pre = pre + jnp.where(jnp.broadcast_to(s2 < sid, (16,)), row, 0)
    own_half = sbuf[pl.ds(16, 16)]
    pre = pre + jnp.where(jnp.broadcast_to(cid == 1, (16,)), own_half, 0)

    padded = ((cnt + (T - 1)) // T) * T
    incl = plsc.cumsum(padded)
    base = (incl - padded) + pre

    half = PPW // 2
    for c2 in range(PPW // 16):
        ev = eb[pl.ds(cid * PPW + c2 * 16, 16)]
        pos = jnp.zeros((16,), jnp.int32)
        delta = jnp.zeros((16,), jnp.int32)
        for e in range(E):
            m = ev == e
            r = plsc.cumsum(jnp.where(m, 1, 0))
            pos = jnp.where(m, _lane(base, e) + r - 1, pos)
            pc = plsc.all_reduce_population_count(m)
            delta = delta + jnp.where(iota16 == e, pc, 0)
        base = base + delta
        p_glob = wid * PPW + c2 * 16 + iota16
        tok = p_glob // 2
        hi = c2 >= (PPW // 32)
        dst_pos, dst_tok = (posb, tokb) if hi else (posa, toka)
        off = (c2 - (PPW // 32)) * 16 if hi else c2 * 16
        dst_pos[pl.ds(off, 16)] = pos
        dst_tok[pl.ds(off, 16)] = tok

    ipa = pltpu.async_copy(posa, invpos.at[pl.ds(wid * PPW, half)], sem3)
    ga = pltpu.async_copy(x2d.at[toka], rowsa, sem)
    gb = pltpu.async_copy(x2d.at[tokb], rowsb, sem2)
    ga.wait()
    sa = pltpu.async_copy(rowsa, xs.at[posa], sem)
    gb.wait()
    sb = pltpu.async_copy(rowsb, xs.at[posb], sem2)
    ipa.wait()
    ipb = pltpu.async_copy(posb, invpos.at[pl.ds(wid * PPW + half, half)], sem3)
    sa.wait()
    wa = pltpu.async_copy(wb.at[pl.ds(0, half)], wsort.at[posa], sem)
    sb.wait()
    wb2 = pltpu.async_copy(wb.at[pl.ds(half, half)], wsort.at[posb], sem2)
    ipb.wait()
    wa.wait()
    wb2.wait()

    @pl.when(wid == 0)
    def _():
        la = jnp.max(jnp.where((cnt > 0) & (iota16 < E), iota16, 0))
        total = jnp.max(incl)
        s16[...] = jnp.broadcast_to(total // T, (16,))
        pltpu.sync_copy(s16, nact)
        for j in range(3):
            tstart = (j * 16 + iota16) * T
            acc = jnp.zeros((16,), jnp.int32)
            for e in range(E):
                acc = acc + jnp.where(tstart >= _lane(incl, e), 1, 0)
            s16[...] = jnp.minimum(acc, la)
            pltpu.sync_copy(s16, gtile.at[pl.ds(j * 16, 16)])


def _route(eflat, wflat, x2d):
    return pl.kernel(
        _route_body,
        out_type=(
            jax.ShapeDtypeStruct((P, D), jnp.float32),
            jax.ShapeDtypeStruct((P,), jnp.float32),
            jax.ShapeDtypeStruct((N,), jnp.int32),
            jax.ShapeDtypeStruct((48,), jnp.int32),
            jax.ShapeDtypeStruct((16,), jnp.int32),
        ),
        mesh=_mesh(),
        compiler_params=pltpu.CompilerParams(needs_layout_passes=False),
        scratch_types=[
            pltpu.VMEM((N // 16,), jnp.int32),
            pltpu.VMEM((PPW,), jnp.float32),
            pltpu.VMEM((32,), jnp.int32),
            pltpu.VMEM((512,), jnp.int32),
            pltpu.VMEM((PPW // 2,), jnp.int32),
            pltpu.VMEM((PPW // 2,), jnp.int32),
            pltpu.VMEM((PPW // 2,), jnp.int32),
            pltpu.VMEM((PPW // 2,), jnp.int32),
            pltpu.VMEM((PPW // 2, D), jnp.float32),
            pltpu.VMEM((PPW // 2, D), jnp.float32),
            pltpu.VMEM((16,), jnp.int32),
            pltpu.VMEM_SHARED((512,), jnp.int32),
            pltpu.SemaphoreType.DMA,
            pltpu.SemaphoreType.DMA,
            pltpu.SemaphoreType.DMA,
        ],
    )(eflat, wflat, x2d)


def _gemm_body(g_ref, na_ref, xs_ref, wsc_ref, w1_ref, b1_ref, w2_ref,
               b2_ref, out_ref):
    i = pl.program_id(0)

    @pl.when(i < na_ref[0])
    def _():
        xb = xs_ref[...].astype(jnp.bfloat16)
        w1b = w1_ref[0].astype(jnp.bfloat16)
        h = jnp.dot(xb, w1b, preferred_element_type=jnp.float32)
        h = jnp.maximum(h + b1_ref[0], 0.0).astype(jnp.bfloat16)
        w2b = w2_ref[0].astype(jnp.bfloat16)
        o = jnp.dot(h, w2b, preferred_element_type=jnp.float32)
        out_ref[...] = (o + b2_ref[0]) * wsc_ref[...]


def _grouped_gemm(gtile, nact, xs, wsc, w1, b1, w2, b2):
    grid_spec = pltpu.PrefetchScalarGridSpec(
        num_scalar_prefetch=2,
        grid=(NT,),
        in_specs=[
            pl.BlockSpec((T, D), lambda i, g, na: (i, 0)),
            pl.BlockSpec((T, 1), lambda i, g, na: (i, 0)),
            pl.BlockSpec((1, D, F), lambda i, g, na: (g[i], 0, 0)),
            pl.BlockSpec((1, 1, F), lambda i, g, na: (g[i], 0, 0)),
            pl.BlockSpec((1, F, D), lambda i, g, na: (g[i], 0, 0)),
            pl.BlockSpec((1, 1, D), lambda i, g, na: (g[i], 0, 0)),
        ],
        out_specs=pl.BlockSpec((T, D), lambda i, g, na: (i, 0)),
    )
    return pl.pallas_call(
        _gemm_body,
        grid_spec=grid_spec,
        out_shape=jax.ShapeDtypeStruct((P, D), jnp.float32),
    )(gtile, nact, xs, wsc, w1, b1.reshape(E, 1, F), w2, b2.reshape(E, 1, D))


def _combine_body(contrib, invpos, y, idx, ra, rb, sem):
    wid = _wid()
    base = wid * TPW
    hw = TPW // 2
    for hh in range(2):
        pltpu.sync_copy(invpos.at[pl.ds(2 * (base + hh * hw), 2 * hw)], idx)
        pltpu.async_copy(contrib.at[idx], ra, sem).wait()

        def row_body(r, _):
            for u in range(D // 16):
                sl = pl.ds(u * 16, 16)
                rb[r, sl] = ra[2 * r, sl] + ra[2 * r + 1, sl]
            return 0

        lax.fori_loop(0, hw, row_body, 0)
        pltpu.sync_copy(rb, y.at[pl.ds(base + hh * hw, hw)])


def _combine(contrib, invpos):
    return pl.kernel(
        _combine_body,
        out_type=jax.ShapeDtypeStruct((S, D), jnp.float32),
        mesh=_mesh(),
        compiler_params=pltpu.CompilerParams(needs_layout_passes=False),
        scratch_types=[
            pltpu.VMEM((TPW,), jnp.int32),
            pltpu.VMEM((TPW, D), jnp.float32),
            pltpu.VMEM((TPW // 2, D), jnp.float32),
            pltpu.SemaphoreType.DMA,
        ],
    )(contrib, invpos)


def kernel(x, gate_w, gate_b, w1, b1, w2, b2):
    x2d = x.reshape(S, D)
    e_sk, w_sk = _gating(x2d, gate_w, gate_b)
    eflat = e_sk.reshape(N)
    wflat = w_sk.reshape(N)
    xs, wsort, invpos, gtile, nact = _route(eflat, wflat, x2d)
    contrib = _grouped_gemm(gtile, nact, xs, wsort.reshape(P, 1),
                            w1, b1, w2, b2)
    y = _combine(contrib, invpos)
    return y.reshape(B, S, D)


# speedup vs baseline: 1.0125x; 1.0125x over previous
import functools

import jax
import jax.numpy as jnp
from jax import lax
from jax.experimental import pallas as pl
from jax.experimental.pallas import tpu as pltpu
from jax.experimental.pallas import tpu_sc as plsc

B, S, D, E, F, K = 1, 2048, 768, 8, 2048, 2
D2 = D // 2
N = S * K
T = 128
P = N + E * T
NT = P // T
NW = 32
PPW = N // NW
SPW = P // NW
TPW = S // NW

_mesh = functools.partial(
    plsc.VectorSubcoreMesh, core_axis_name="c", subcore_axis_name="s",
    num_cores=2, num_subcores=16)


def _wid():
    return lax.axis_index("s") * 2 + lax.axis_index("c")


def _gate_body(x_ref, gw_ref, gb_ref, e_ref, w_ref):
    x = x_ref[...]
    logits = jnp.dot(x, gw_ref[...], preferred_element_type=jnp.float32)
    logits = logits + gb_ref[...]
    iota = lax.broadcasted_iota(jnp.int32, (S, E), 1)
    m0 = jnp.max(logits, axis=1, keepdims=True)
    e0 = jnp.min(jnp.where(logits == m0, iota, E), axis=1, keepdims=True)
    mask0 = iota == e0
    l1m = jnp.where(mask0, jnp.float32(-1e30), logits)
    m1 = jnp.max(l1m, axis=1, keepdims=True)
    e1 = jnp.min(jnp.where(l1m == m1, iota, E), axis=1, keepdims=True)
    a = jnp.exp(m1 - m0)
    w0 = 1.0 / (1.0 + a)
    e_ref[...] = jnp.concatenate([e0, e1], axis=1)
    w_ref[...] = jnp.concatenate([w0, 1.0 - w0], axis=1)


def _gating(x2d, gate_w, gate_b):
    return pl.pallas_call(
        _gate_body,
        out_shape=(
            jax.ShapeDtypeStruct((S, K), jnp.int32),
            jax.ShapeDtypeStruct((S, K), jnp.float32),
        ),
    )(x2d, gate_w, gate_b.reshape(1, E))


def _lane(vec, e):
    iota16 = lax.iota(jnp.int32, 16)
    return jnp.max(jnp.where(iota16 == e, vec, jnp.int32(-2147483647)))


def _route_body(eflat, wflat, x2d, xs, wsort, invpos, gtile, nact,
                eb, wb, sbuf, shv, posa, posb, toka, tokb,
                rowsa, rowsb, s16, sh, sem, sem2, sem3):
    sid = lax.axis_index("s")
    cid = lax.axis_index("c")
    wid = sid * 2 + cid
    iota16 = lax.iota(jnp.int32, 16)
    blk = N // 16
    pltpu.sync_copy(eflat.at[pl.ds(sid * blk, blk)], eb)
    pltpu.sync_copy(wflat.at[pl.ds(wid * PPW, PPW)], wb)

    loc = jnp.zeros((16,), jnp.int32)
    hlf = jnp.zeros((16,), jnp.int32)
    for cc in range(blk // 16):
        if cc == blk // 32:
            hlf = loc
        ev = eb[pl.ds(cc * 16, 16)]
        upd = jnp.zeros((16,), jnp.int32)
        for e in range(E):
            pc = plsc.all_reduce_population_count(ev == e)
            upd = upd + jnp.where(iota16 == e, pc, 0)
        loc = loc + upd
    sbuf[pl.ds(0, 16)] = loc
    sbuf[pl.ds(16, 16)] = hlf
    pltpu.sync_copy(sbuf, sh.at[pl.ds(sid * 32, 32)])
    plsc.subcore_barrier()
    pltpu.sync_copy(sh, shv)

    cnt = jnp.zeros((16,), jnp.int32)
    pre = jnp.zeros((16,), jnp.int32)
    for s2 in range(16):
        row = shv[pl.ds(s2 * 32, 16)]
        cnt = cnt + row
        pre = pre + jnp.where(jnp.broadcast_to(s2 < sid, (16,)), row, 0)
    own_half = sbuf[pl.ds(16, 16)]
    pre = pre + jnp.where(jnp.broadcast_to(cid == 1, (16,)), own_half, 0)

    padded = ((cnt + (T - 1)) // T) * T
    incl = plsc.cumsum(padded)
    base = (incl - padded) + pre

    half = PPW // 2
    for c2 in range(PPW // 16):
        ev = eb[pl.ds(cid * PPW + c2 * 16, 16)]
        pos = jnp.zeros((16,), jnp.int32)
        delta = jnp.zeros((16,), jnp.int32)
        for e in range(E):
            m = ev == e
            r = plsc.cumsum(jnp.where(m, 1, 0))
            pos = jnp.where(m, _lane(base, e) + r - 1, pos)
            pc = plsc.all_reduce_population_count(m)
            delta = delta + jnp.where(iota16 == e, pc, 0)
        base = base + delta
        p_glob = wid * PPW + c2 * 16 + iota16
        tok = p_glob // 2
        hi = c2 >= (PPW // 32)
        dst_pos, dst_tok = (posb, tokb) if hi else (posa, toka)
        off = (c2 - (PPW // 32)) * 16 if hi else c2 * 16
        dst_pos[pl.ds(off, 16)] = pos
        dst_tok[pl.ds(off, 16)] = tok

    ipa = pltpu.async_copy(posa, invpos.at[pl.ds(wid * PPW, half)], sem3)
    ga = pltpu.async_copy(x2d.at[toka], rowsa, sem)
    gb = pltpu.async_copy(x2d.at[tokb], rowsb, sem2)
    ga.wait()
    sa = pltpu.async_copy(rowsa, xs.at[posa], sem)
    gb.wait()
    sb = pltpu.async_copy(rowsb, xs.at[posb], sem2)
    ipa.wait()
    ipb = pltpu.async_copy(posb, invpos.at[pl.ds(wid * PPW + half, half)], sem3)
    sa.wait()
    wa = pltpu.async_copy(wb.at[pl.ds(0, half)], wsort.at[posa], sem)
    sb.wait()
    wb2 = pltpu.async_copy(wb.at[pl.ds(half, half)], wsort.at[posb], sem2)
    ipb.wait()
    wa.wait()
    wb2.wait()

    @pl.when(wid == 0)
    def _():
        la = jnp.max(jnp.where((cnt > 0) & (iota16 < E), iota16, 0))
        total = jnp.max(incl)
        s16[...] = jnp.broadcast_to(total // T, (16,))
        pltpu.sync_copy(s16, nact)
        for j in range(3):
            tstart = (j * 16 + iota16) * T
            acc = jnp.zeros((16,), jnp.int32)
            for e in range(E):
                acc = acc + jnp.where(tstart >= _lane(incl, e), 1, 0)
            s16[...] = jnp.minimum(acc, la)
            pltpu.sync_copy(s16, gtile.at[pl.ds(j * 16, 16)])


def _route(eflat, wflat, x2d):
    return pl.kernel(
        _route_body,
        out_type=(
            jax.ShapeDtypeStruct((P, D), jnp.float32),
            jax.ShapeDtypeStruct((P,), jnp.float32),
            jax.ShapeDtypeStruct((N,), jnp.int32),
            jax.ShapeDtypeStruct((48,), jnp.int32),
            jax.ShapeDtypeStruct((16,), jnp.int32),
        ),
        mesh=_mesh(),
        compiler_params=pltpu.CompilerParams(needs_layout_passes=False),
        scratch_types=[
            pltpu.VMEM((N // 16,), jnp.int32),
            pltpu.VMEM((PPW,), jnp.float32),
            pltpu.VMEM((32,), jnp.int32),
            pltpu.VMEM((512,), jnp.int32),
            pltpu.VMEM((PPW // 2,), jnp.int32),
            pltpu.VMEM((PPW // 2,), jnp.int32),
            pltpu.VMEM((PPW // 2,), jnp.int32),
            pltpu.VMEM((PPW // 2,), jnp.int32),
            pltpu.VMEM((PPW // 2, D), jnp.float32),
            pltpu.VMEM((PPW // 2, D), jnp.float32),
            pltpu.VMEM((16,), jnp.int32),
            pltpu.VMEM_SHARED((512,), jnp.int32),
            pltpu.SemaphoreType.DMA,
            pltpu.SemaphoreType.DMA,
            pltpu.SemaphoreType.DMA,
        ],
    )(eflat, wflat, x2d)


def _gemm_body(g_ref, na_ref, xs_ref, wsc_ref, w1_ref, b1_ref, w2_ref,
               b2_ref, out_ref):
    i = pl.program_id(0)

    @pl.when(i < na_ref[0])
    def _():
        xb = xs_ref[...].astype(jnp.bfloat16)
        w1b = w1_ref[0].astype(jnp.bfloat16)
        h = jnp.dot(xb, w1b, preferred_element_type=jnp.float32)
        h = jnp.maximum(h + b1_ref[0], 0.0).astype(jnp.bfloat16)
        w2b = w2_ref[0].astype(jnp.bfloat16)
        o = jnp.dot(h, w2b, preferred_element_type=jnp.float32)
        out_ref[...] = (o + b2_ref[0]) * wsc_ref[...]


def _grouped_gemm(gtile, nact, xs, wsc, w1, b1, w2, b2):
    grid_spec = pltpu.PrefetchScalarGridSpec(
        num_scalar_prefetch=2,
        grid=(NT,),
        in_specs=[
            pl.BlockSpec((T, D), lambda i, g, na: (i, 0)),
            pl.BlockSpec((T, 1), lambda i, g, na: (i, 0)),
            pl.BlockSpec((1, D, F), lambda i, g, na: (g[i], 0, 0)),
            pl.BlockSpec((1, 1, F), lambda i, g, na: (g[i], 0, 0)),
            pl.BlockSpec((1, F, D), lambda i, g, na: (g[i], 0, 0)),
            pl.BlockSpec((1, 1, D), lambda i, g, na: (g[i], 0, 0)),
        ],
        out_specs=pl.BlockSpec((T, D), lambda i, g, na: (i, 0)),
    )
    return pl.pallas_call(
        _gemm_body,
        grid_spec=grid_spec,
        out_shape=jax.ShapeDtypeStruct((P, D), jnp.float32),
    )(gtile, nact, xs, wsc, w1, b1.reshape(E, 1, F), w2, b2.reshape(E, 1, D))


def _combine_body(contrib, invpos, y, idx, idx2, ra, ra2, sem, sem2):
    wid = _wid()
    base = wid * TPW
    hw = TPW // 2
    pltpu.sync_copy(invpos.at[pl.ds(2 * base, 2 * hw)], idx)
    ga = pltpu.async_copy(contrib.at[idx], ra, sem)
    pltpu.sync_copy(invpos.at[pl.ds(2 * base + TPW, 2 * hw)], idx2)
    gb = pltpu.async_copy(contrib.at[idx2], ra2, sem2)

    def make_row_body(buf):
        def row_body(r, _):
            for u in range(D // 16):
                sl = pl.ds(u * 16, 16)
                buf[r, sl] = buf[2 * r, sl] + buf[2 * r + 1, sl]
            return 0
        return row_body

    ga.wait()
    lax.fori_loop(0, hw, make_row_body(ra), 0)
    oa = pltpu.async_copy(ra.at[pl.ds(0, hw)], y.at[pl.ds(base, hw)], sem)
    gb.wait()
    lax.fori_loop(0, hw, make_row_body(ra2), 0)
    ob = pltpu.async_copy(ra2.at[pl.ds(0, hw)], y.at[pl.ds(base + hw, hw)], sem2)
    oa.wait()
    ob.wait()


def _combine(contrib, invpos):
    return pl.kernel(
        _combine_body,
        out_type=jax.ShapeDtypeStruct((S, D), jnp.float32),
        mesh=_mesh(),
        compiler_params=pltpu.CompilerParams(needs_layout_passes=False),
        scratch_types=[
            pltpu.VMEM((TPW,), jnp.int32),
            pltpu.VMEM((TPW,), jnp.int32),
            pltpu.VMEM((TPW, D), jnp.float32),
            pltpu.VMEM((TPW, D), jnp.float32),
            pltpu.SemaphoreType.DMA,
            pltpu.SemaphoreType.DMA,
        ],
    )(contrib, invpos)


def kernel(x, gate_w, gate_b, w1, b1, w2, b2):
    x2d = x.reshape(S, D)
    e_sk, w_sk = _gating(x2d, gate_w, gate_b)
    eflat = e_sk.reshape(N)
    wflat = w_sk.reshape(N)
    xs, wsort, invpos, gtile, nact = _route(eflat, wflat, x2d)
    contrib = _grouped_gemm(gtile, nact, xs, wsort.reshape(P, 1),
                            w1, b1, w2, b2)
    y = _combine(contrib, invpos)
    return y.reshape(B, S, D)


# speedup vs baseline: 1.0818x; 1.0684x over previous
import functools

import jax
import jax.numpy as jnp
from jax import lax
from jax.experimental import pallas as pl
from jax.experimental.pallas import tpu as pltpu
from jax.experimental.pallas import tpu_sc as plsc

B, S, D, E, F, K = 1, 2048, 768, 8, 2048, 2
D2 = D // 2
N = S * K
T = 128
P = N + E * T
NT = P // T
NW = 32
PPW = N // NW
SPW = P // NW
TPW = S // NW

_mesh = functools.partial(
    plsc.VectorSubcoreMesh, core_axis_name="c", subcore_axis_name="s",
    num_cores=2, num_subcores=16)


def _wid():
    return lax.axis_index("s") * 2 + lax.axis_index("c")


def _gate_body(x_ref, gw_ref, gb_ref, e_ref, w_ref):
    x = x_ref[...]
    logits = jnp.dot(x, gw_ref[...], preferred_element_type=jnp.float32)
    logits = logits + gb_ref[...]
    iota = lax.broadcasted_iota(jnp.int32, (S, E), 1)
    m0 = jnp.max(logits, axis=1, keepdims=True)
    e0 = jnp.min(jnp.where(logits == m0, iota, E), axis=1, keepdims=True)
    mask0 = iota == e0
    l1m = jnp.where(mask0, jnp.float32(-1e30), logits)
    m1 = jnp.max(l1m, axis=1, keepdims=True)
    e1 = jnp.min(jnp.where(l1m == m1, iota, E), axis=1, keepdims=True)
    a = jnp.exp(m1 - m0)
    w0 = 1.0 / (1.0 + a)
    e_ref[...] = jnp.concatenate([e0, e1], axis=1)
    w_ref[...] = jnp.concatenate([w0, 1.0 - w0], axis=1)


def _gating(x2d, gate_w, gate_b):
    return pl.pallas_call(
        _gate_body,
        out_shape=(
            jax.ShapeDtypeStruct((S, K), jnp.int32),
            jax.ShapeDtypeStruct((S, K), jnp.float32),
        ),
    )(x2d, gate_w, gate_b.reshape(1, E))


def _lane(vec, e):
    iota16 = lax.iota(jnp.int32, 16)
    return jnp.max(jnp.where(iota16 == e, vec, jnp.int32(-2147483647)))


def _route_body(eflat, wflat, x2d, xs, wsort, invpos, gtile, nact,
                eb, wb, sbuf, shv, posa, posb, toka, tokb,
                rowsa, rowsb, s16, sh, sem, sem2, sem3):
    sid = lax.axis_index("s")
    cid = lax.axis_index("c")
    wid = sid * 2 + cid
    iota16 = lax.iota(jnp.int32, 16)
    blk = N // 16
    pltpu.sync_copy(eflat.at[pl.ds(sid * blk, blk)], eb)
    pltpu.sync_copy(wflat.at[pl.ds(wid * PPW, PPW)], wb)

    loc = jnp.zeros((16,), jnp.int32)
    hlf = jnp.zeros((16,), jnp.int32)
    for cc in range(blk // 16):
        if cc == blk // 32:
            hlf = loc
        ev = eb[pl.ds(cc * 16, 16)]
        upd = jnp.zeros((16,), jnp.int32)
        for e in range(E):
            pc = plsc.all_reduce_population_count(ev == e)
            upd = upd + jnp.where(iota16 == e, pc, 0)
        loc = loc + upd
    sbuf[pl.ds(0, 16)] = loc
    sbuf[pl.ds(16, 16)] = hlf
    pltpu.sync_copy(sbuf, sh.at[pl.ds(sid * 32, 32)])
    plsc.subcore_barrier()
    pltpu.sync_copy(sh, shv)

    cnt = jnp.zeros((16,), jnp.int32)
    pre = jnp.zeros((16,), jnp.int32)
    for s2 in range(16):
        row = shv[pl.ds(s2 * 32, 16)]
        cnt = cnt + row
        pre = pre + jnp.where(jnp.broadcast_to(s2 < sid, (16,)), row, 0)
    own_half = sbuf[pl.ds(16, 16)]
    pre = pre + jnp.where(jnp.broadcast_to(cid == 1, (16,)), own_half, 0)

    padded = ((cnt + (T - 1)) // T) * T
    incl = plsc.cumsum(padded)
    base = (incl - padded) + pre

    half = PPW // 2
    for c2 in range(PPW // 16):
        ev = eb[pl.ds(cid * PPW + c2 * 16, 16)]
        pos = jnp.zeros((16,), jnp.int32)
        delta = jnp.zeros((16,), jnp.int32)
        for e in range(E):
            m = ev == e
            r = plsc.cumsum(jnp.where(m, 1, 0))
            pos = jnp.where(m, _lane(base, e) + r - 1, pos)
            pc = plsc.all_reduce_population_count(m)
            delta = delta + jnp.where(iota16 == e, pc, 0)
        base = base + delta
        p_glob = wid * PPW + c2 * 16 + iota16
        tok = p_glob - jnp.where(p_glob >= S, S, 0)
        hi = c2 >= (PPW // 32)
        dst_pos, dst_tok = (posb, tokb) if hi else (posa, toka)
        off = (c2 - (PPW // 32)) * 16 if hi else c2 * 16
        dst_pos[pl.ds(off, 16)] = pos
        dst_tok[pl.ds(off, 16)] = tok

    ipa = pltpu.async_copy(posa, invpos.at[pl.ds(wid * PPW, half)], sem3)
    ga = pltpu.async_copy(x2d.at[toka], rowsa, sem)
    gb = pltpu.async_copy(x2d.at[tokb], rowsb, sem2)
    ga.wait()
    sa = pltpu.async_copy(rowsa, xs.at[posa], sem)
    gb.wait()
    sb = pltpu.async_copy(rowsb, xs.at[posb], sem2)
    ipa.wait()
    ipb = pltpu.async_copy(posb, invpos.at[pl.ds(wid * PPW + half, half)], sem3)
    sa.wait()
    wa = pltpu.async_copy(wb.at[pl.ds(0, half)], wsort.at[posa], sem)
    sb.wait()
    wb2 = pltpu.async_copy(wb.at[pl.ds(half, half)], wsort.at[posb], sem2)
    ipb.wait()
    wa.wait()
    wb2.wait()

    @pl.when(wid == 0)
    def _():
        la = jnp.max(jnp.where((cnt > 0) & (iota16 < E), iota16, 0))
        total = jnp.max(incl)
        s16[...] = jnp.broadcast_to(total // T, (16,))
        pltpu.sync_copy(s16, nact)
        for j in range(3):
            tstart = (j * 16 + iota16) * T
            acc = jnp.zeros((16,), jnp.int32)
            for e in range(E):
                acc = acc + jnp.where(tstart >= _lane(incl, e), 1, 0)
            s16[...] = jnp.minimum(acc, la)
            pltpu.sync_copy(s16, gtile.at[pl.ds(j * 16, 16)])


def _route(eflat, wflat, x2d):
    return pl.kernel(
        _route_body,
        out_type=(
            jax.ShapeDtypeStruct((P, D), jnp.float32),
            jax.ShapeDtypeStruct((P,), jnp.float32),
            jax.ShapeDtypeStruct((N,), jnp.int32),
            jax.ShapeDtypeStruct((48,), jnp.int32),
            jax.ShapeDtypeStruct((16,), jnp.int32),
        ),
        mesh=_mesh(),
        compiler_params=pltpu.CompilerParams(needs_layout_passes=False),
        scratch_types=[
            pltpu.VMEM((N // 16,), jnp.int32),
            pltpu.VMEM((PPW,), jnp.float32),
            pltpu.VMEM((32,), jnp.int32),
            pltpu.VMEM((512,), jnp.int32),
            pltpu.VMEM((PPW // 2,), jnp.int32),
            pltpu.VMEM((PPW // 2,), jnp.int32),
            pltpu.VMEM((PPW // 2,), jnp.int32),
            pltpu.VMEM((PPW // 2,), jnp.int32),
            pltpu.VMEM((PPW // 2, D), jnp.float32),
            pltpu.VMEM((PPW // 2, D), jnp.float32),
            pltpu.VMEM((16,), jnp.int32),
            pltpu.VMEM_SHARED((512,), jnp.int32),
            pltpu.SemaphoreType.DMA,
            pltpu.SemaphoreType.DMA,
            pltpu.SemaphoreType.DMA,
        ],
    )(eflat, wflat, x2d)


def _gemm_body(g_ref, na_ref, xs_ref, wsc_ref, w1_ref, b1_ref, w2_ref,
               b2_ref, out_ref):
    i = pl.program_id(0)

    @pl.when(i < na_ref[0])
    def _():
        xb = xs_ref[...].astype(jnp.bfloat16)
        w1b = w1_ref[0].astype(jnp.bfloat16)
        h = jnp.dot(xb, w1b, preferred_element_type=jnp.float32)
        h = jnp.maximum(h + b1_ref[0], 0.0).astype(jnp.bfloat16)
        w2b = w2_ref[0].astype(jnp.bfloat16)
        o = jnp.dot(h, w2b, preferred_element_type=jnp.float32)
        out_ref[...] = (o + b2_ref[0]) * wsc_ref[...]


def _grouped_gemm(gtile, nact, xs, wsc, w1, b1, w2, b2):
    grid_spec = pltpu.PrefetchScalarGridSpec(
        num_scalar_prefetch=2,
        grid=(NT,),
        in_specs=[
            pl.BlockSpec((T, D), lambda i, g, na: (i, 0)),
            pl.BlockSpec((T, 1), lambda i, g, na: (i, 0)),
            pl.BlockSpec((1, D, F), lambda i, g, na: (g[i], 0, 0)),
            pl.BlockSpec((1, 1, F), lambda i, g, na: (g[i], 0, 0)),
            pl.BlockSpec((1, F, D), lambda i, g, na: (g[i], 0, 0)),
            pl.BlockSpec((1, 1, D), lambda i, g, na: (g[i], 0, 0)),
        ],
        out_specs=pl.BlockSpec((T, D), lambda i, g, na: (i, 0)),
    )
    return pl.pallas_call(
        _gemm_body,
        grid_spec=grid_spec,
        out_shape=jax.ShapeDtypeStruct((P, D), jnp.float32),
    )(gtile, nact, xs, wsc, w1, b1.reshape(E, 1, F), w2, b2.reshape(E, 1, D))


def _combine_body(contrib, invpos, y, idx, idx2, ra, ra2, sem, sem2):
    wid = _wid()
    base = wid * TPW
    pltpu.sync_copy(invpos.at[pl.ds(base, TPW)], idx)
    ga = pltpu.async_copy(contrib.at[idx], ra, sem)
    pltpu.sync_copy(invpos.at[pl.ds(S + base, TPW)], idx2)
    gb = pltpu.async_copy(contrib.at[idx2], ra2, sem2)
    ga.wait()
    gb.wait()

    def row_body(r, _):
        for u in range(D // 16):
            sl = pl.ds(u * 16, 16)
            ra[r, sl] = ra[r, sl] + ra2[r, sl]
        return 0

    lax.fori_loop(0, TPW, row_body, 0)
    pltpu.sync_copy(ra, y.at[pl.ds(base, TPW)])


def _combine(contrib, invpos):
    return pl.kernel(
        _combine_body,
        out_type=jax.ShapeDtypeStruct((S, D), jnp.float32),
        mesh=_mesh(),
        compiler_params=pltpu.CompilerParams(needs_layout_passes=False),
        scratch_types=[
            pltpu.VMEM((TPW,), jnp.int32),
            pltpu.VMEM((TPW,), jnp.int32),
            pltpu.VMEM((TPW, D), jnp.float32),
            pltpu.VMEM((TPW, D), jnp.float32),
            pltpu.SemaphoreType.DMA,
            pltpu.SemaphoreType.DMA,
        ],
    )(contrib, invpos)


def kernel(x, gate_w, gate_b, w1, b1, w2, b2):
    x2d = x.reshape(S, D)
    e_sk, w_sk = _gating(x2d, gate_w, gate_b)
    eflat = e_sk.T.reshape(N)
    wflat = w_sk.T.reshape(N)
    xs, wsort, invpos, gtile, nact = _route(eflat, wflat, x2d)
    contrib = _grouped_gemm(gtile, nact, xs, wsort.reshape(P, 1),
                            w1, b1, w2, b2)
    y = _combine(contrib, invpos)
    return y.reshape(B, S, D)


# speedup vs baseline: 1.1497x; 1.0627x over previous
import functools

import jax
import jax.numpy as jnp
from jax import lax
from jax.experimental import pallas as pl
from jax.experimental.pallas import tpu as pltpu
from jax.experimental.pallas import tpu_sc as plsc

B, S, D, E, F, K = 1, 2048, 768, 8, 2048, 2
D2 = D // 2
N = S * K
T = 256
P = N + E * T
NT = P // T
NW = 32
PPW = N // NW
SPW = P // NW
TPW = S // NW

_mesh = functools.partial(
    plsc.VectorSubcoreMesh, core_axis_name="c", subcore_axis_name="s",
    num_cores=2, num_subcores=16)


def _wid():
    return lax.axis_index("s") * 2 + lax.axis_index("c")


def _gate_body(x_ref, gw_ref, gb_ref, e_ref, w_ref):
    x = x_ref[...]
    logits = jnp.dot(x, gw_ref[...], preferred_element_type=jnp.float32)
    logits = logits + gb_ref[...]
    iota = lax.broadcasted_iota(jnp.int32, (S, E), 1)
    m0 = jnp.max(logits, axis=1, keepdims=True)
    e0 = jnp.min(jnp.where(logits == m0, iota, E), axis=1, keepdims=True)
    mask0 = iota == e0
    l1m = jnp.where(mask0, jnp.float32(-1e30), logits)
    m1 = jnp.max(l1m, axis=1, keepdims=True)
    e1 = jnp.min(jnp.where(l1m == m1, iota, E), axis=1, keepdims=True)
    a = jnp.exp(m1 - m0)
    w0 = 1.0 / (1.0 + a)
    e_ref[...] = jnp.concatenate([e0, e1], axis=1)
    w_ref[...] = jnp.concatenate([w0, 1.0 - w0], axis=1)


def _gating(x2d, gate_w, gate_b):
    return pl.pallas_call(
        _gate_body,
        out_shape=(
            jax.ShapeDtypeStruct((S, K), jnp.int32),
            jax.ShapeDtypeStruct((S, K), jnp.float32),
        ),
    )(x2d, gate_w, gate_b.reshape(1, E))


def _lane(vec, e):
    iota16 = lax.iota(jnp.int32, 16)
    return jnp.max(jnp.where(iota16 == e, vec, jnp.int32(-2147483647)))


def _route_body(eflat, wflat, x2d, xs, wsort, invpos, gtile, nact,
                eb, wb, sbuf, shv, posa, posb, toka, tokb,
                rowsa, rowsb, s16, sh, sem, sem2, sem3):
    sid = lax.axis_index("s")
    cid = lax.axis_index("c")
    wid = sid * 2 + cid
    iota16 = lax.iota(jnp.int32, 16)
    blk = N // 16
    pltpu.sync_copy(eflat.at[pl.ds(sid * blk, blk)], eb)
    pltpu.sync_copy(wflat.at[pl.ds(wid * PPW, PPW)], wb)

    loc = jnp.zeros((16,), jnp.int32)
    hlf = jnp.zeros((16,), jnp.int32)
    for cc in range(blk // 16):
        if cc == blk // 32:
            hlf = loc
        ev = eb[pl.ds(cc * 16, 16)]
        upd = jnp.zeros((16,), jnp.int32)
        for e in range(E):
            pc = plsc.all_reduce_population_count(ev == e)
            upd = upd + jnp.where(iota16 == e, pc, 0)
        loc = loc + upd
    sbuf[pl.ds(0, 16)] = loc
    sbuf[pl.ds(16, 16)] = hlf
    pltpu.sync_copy(sbuf, sh.at[pl.ds(sid * 32, 32)])
    plsc.subcore_barrier()
    pltpu.sync_copy(sh, shv)

    cnt = jnp.zeros((16,), jnp.int32)
    pre = jnp.zeros((16,), jnp.int32)
    for s2 in range(16):
        row = shv[pl.ds(s2 * 32, 16)]
        cnt = cnt + row
        pre = pre + jnp.where(jnp.broadcast_to(s2 < sid, (16,)), row, 0)
    own_half = sbuf[pl.ds(16, 16)]
    pre = pre + jnp.where(jnp.broadcast_to(cid == 1, (16,)), own_half, 0)

    padded = ((cnt + (T - 1)) // T) * T
    incl = plsc.cumsum(padded)
    base = (incl - padded) + pre

    half = PPW // 2
    for c2 in range(PPW // 16):
        ev = eb[pl.ds(cid * PPW + c2 * 16, 16)]
        pos = jnp.zeros((16,), jnp.int32)
        delta = jnp.zeros((16,), jnp.int32)
        for e in range(E):
            m = ev == e
            r = plsc.cumsum(jnp.where(m, 1, 0))
            pos = jnp.where(m, _lane(base, e) + r - 1, pos)
            pc = plsc.all_reduce_population_count(m)
            delta = delta + jnp.where(iota16 == e, pc, 0)
        base = base + delta
        p_glob = wid * PPW + c2 * 16 + iota16
        tok = p_glob - jnp.where(p_glob >= S, S, 0)
        hi = c2 >= (PPW // 32)
        dst_pos, dst_tok = (posb, tokb) if hi else (posa, toka)
        off = (c2 - (PPW // 32)) * 16 if hi else c2 * 16
        dst_pos[pl.ds(off, 16)] = pos
        dst_tok[pl.ds(off, 16)] = tok

    ipa = pltpu.async_copy(posa, invpos.at[pl.ds(wid * PPW, half)], sem3)
    ga = pltpu.async_copy(x2d.at[toka], rowsa, sem)
    gb = pltpu.async_copy(x2d.at[tokb], rowsb, sem2)
    ga.wait()
    sa = pltpu.async_copy(rowsa, xs.at[posa], sem)
    gb.wait()
    sb = pltpu.async_copy(rowsb, xs.at[posb], sem2)
    ipa.wait()
    ipb = pltpu.async_copy(posb, invpos.at[pl.ds(wid * PPW + half, half)], sem3)
    sa.wait()
    wa = pltpu.async_copy(wb.at[pl.ds(0, half)], wsort.at[posa], sem)
    sb.wait()
    wb2 = pltpu.async_copy(wb.at[pl.ds(half, half)], wsort.at[posb], sem2)
    ipb.wait()
    wa.wait()
    wb2.wait()

    @pl.when(wid == 0)
    def _():
        la = jnp.max(jnp.where((cnt > 0) & (iota16 < E), iota16, 0))
        total = jnp.max(incl)
        s16[...] = jnp.broadcast_to(total // T, (16,))
        pltpu.sync_copy(s16, nact)
        for j in range(3):
            tstart = (j * 16 + iota16) * T
            acc = jnp.zeros((16,), jnp.int32)
            for e in range(E):
                acc = acc + jnp.where(tstart >= _lane(incl, e), 1, 0)
            s16[...] = jnp.minimum(acc, la)
            pltpu.sync_copy(s16, gtile.at[pl.ds(j * 16, 16)])


def _route(eflat, wflat, x2d):
    return pl.kernel(
        _route_body,
        out_type=(
            jax.ShapeDtypeStruct((P, D), jnp.float32),
            jax.ShapeDtypeStruct((P,), jnp.float32),
            jax.ShapeDtypeStruct((N,), jnp.int32),
            jax.ShapeDtypeStruct((48,), jnp.int32),
            jax.ShapeDtypeStruct((16,), jnp.int32),
        ),
        mesh=_mesh(),
        compiler_params=pltpu.CompilerParams(needs_layout_passes=False),
        scratch_types=[
            pltpu.VMEM((N // 16,), jnp.int32),
            pltpu.VMEM((PPW,), jnp.float32),
            pltpu.VMEM((32,), jnp.int32),
            pltpu.VMEM((512,), jnp.int32),
            pltpu.VMEM((PPW // 2,), jnp.int32),
            pltpu.VMEM((PPW // 2,), jnp.int32),
            pltpu.VMEM((PPW // 2,), jnp.int32),
            pltpu.VMEM((PPW // 2,), jnp.int32),
            pltpu.VMEM((PPW // 2, D), jnp.float32),
            pltpu.VMEM((PPW // 2, D), jnp.float32),
            pltpu.VMEM((16,), jnp.int32),
            pltpu.VMEM_SHARED((512,), jnp.int32),
            pltpu.SemaphoreType.DMA,
            pltpu.SemaphoreType.DMA,
            pltpu.SemaphoreType.DMA,
        ],
    )(eflat, wflat, x2d)


def _gemm_body(g_ref, na_ref, xs_ref, wsc_ref, w1_ref, b1_ref, w2_ref,
               b2_ref, out_ref):
    i = pl.program_id(0)

    @pl.when(i < na_ref[0])
    def _():
        xb = xs_ref[...].astype(jnp.bfloat16)
        w1b = w1_ref[0].astype(jnp.bfloat16)
        h = jnp.dot(xb, w1b, preferred_element_type=jnp.float32)
        h = jnp.maximum(h + b1_ref[0], 0.0).astype(jnp.bfloat16)
        w2b = w2_ref[0].astype(jnp.bfloat16)
        o = jnp.dot(h, w2b, preferred_element_type=jnp.float32)
        out_ref[...] = (o + b2_ref[0]) * wsc_ref[...]


def _grouped_gemm(gtile, nact, xs, wsc, w1, b1, w2, b2):
    grid_spec = pltpu.PrefetchScalarGridSpec(
        num_scalar_prefetch=2,
        grid=(NT,),
        in_specs=[
            pl.BlockSpec((T, D), lambda i, g, na: (i, 0)),
            pl.BlockSpec((T, 1), lambda i, g, na: (i, 0)),
            pl.BlockSpec((1, D, F), lambda i, g, na: (g[i], 0, 0)),
            pl.BlockSpec((1, 1, F), lambda i, g, na: (g[i], 0, 0)),
            pl.BlockSpec((1, F, D), lambda i, g, na: (g[i], 0, 0)),
            pl.BlockSpec((1, 1, D), lambda i, g, na: (g[i], 0, 0)),
        ],
        out_specs=pl.BlockSpec((T, D), lambda i, g, na: (i, 0)),
    )
    return pl.pallas_call(
        _gemm_body,
        grid_spec=grid_spec,
        out_shape=jax.ShapeDtypeStruct((P, D), jnp.float32),
    )(gtile, nact, xs, wsc, w1, b1.reshape(E, 1, F), w2, b2.reshape(E, 1, D))


def _combine_body(contrib, invpos, y, idx, idx2, ra, ra2, sem, sem2):
    wid = _wid()
    base = wid * TPW
    pltpu.sync_copy(invpos.at[pl.ds(base, TPW)], idx)
    ga = pltpu.async_copy(contrib.at[idx], ra, sem)
    pltpu.sync_copy(invpos.at[pl.ds(S + base, TPW)], idx2)
    gb = pltpu.async_copy(contrib.at[idx2], ra2, sem2)
    ga.wait()
    gb.wait()

    def row_body(r, _):
        for u in range(D // 16):
            sl = pl.ds(u * 16, 16)
            ra[r, sl] = ra[r, sl] + ra2[r, sl]
        return 0

    lax.fori_loop(0, TPW, row_body, 0)
    pltpu.sync_copy(ra, y.at[pl.ds(base, TPW)])


def _combine(contrib, invpos):
    return pl.kernel(
        _combine_body,
        out_type=jax.ShapeDtypeStruct((S, D), jnp.float32),
        mesh=_mesh(),
        compiler_params=pltpu.CompilerParams(needs_layout_passes=False),
        scratch_types=[
            pltpu.VMEM((TPW,), jnp.int32),
            pltpu.VMEM((TPW,), jnp.int32),
            pltpu.VMEM((TPW, D), jnp.float32),
            pltpu.VMEM((TPW, D), jnp.float32),
            pltpu.SemaphoreType.DMA,
            pltpu.SemaphoreType.DMA,
        ],
    )(contrib, invpos)


def kernel(x, gate_w, gate_b, w1, b1, w2, b2):
    x2d = x.reshape(S, D)
    e_sk, w_sk = _gating(x2d, gate_w, gate_b)
    eflat = e_sk.T.reshape(N)
    wflat = w_sk.T.reshape(N)
    xs, wsort, invpos, gtile, nact = _route(eflat, wflat, x2d)
    contrib = _grouped_gemm(gtile, nact, xs, wsort.reshape(P, 1),
                            w1, b1, w2, b2)
    y = _combine(contrib, invpos)
    return y.reshape(B, S, D)


# speedup vs baseline: 1.1759x; 1.0228x over previous
import functools

import jax
import jax.numpy as jnp
from jax import lax
from jax.experimental import pallas as pl
from jax.experimental.pallas import tpu as pltpu
from jax.experimental.pallas import tpu_sc as plsc

B, S, D, E, F, K = 1, 2048, 768, 8, 2048, 2
D2 = D // 2
N = S * K
T = 512
P = N + E * T
NT = P // T
NW = 32
PPW = N // NW
SPW = P // NW
TPW = S // NW

_mesh = functools.partial(
    plsc.VectorSubcoreMesh, core_axis_name="c", subcore_axis_name="s",
    num_cores=2, num_subcores=16)


def _wid():
    return lax.axis_index("s") * 2 + lax.axis_index("c")


def _gate_body(x_ref, gw_ref, gb_ref, e_ref, w_ref):
    x = x_ref[...]
    logits = jnp.dot(x, gw_ref[...], preferred_element_type=jnp.float32)
    logits = logits + gb_ref[...]
    iota = lax.broadcasted_iota(jnp.int32, (S, E), 1)
    m0 = jnp.max(logits, axis=1, keepdims=True)
    e0 = jnp.min(jnp.where(logits == m0, iota, E), axis=1, keepdims=True)
    mask0 = iota == e0
    l1m = jnp.where(mask0, jnp.float32(-1e30), logits)
    m1 = jnp.max(l1m, axis=1, keepdims=True)
    e1 = jnp.min(jnp.where(l1m == m1, iota, E), axis=1, keepdims=True)
    a = jnp.exp(m1 - m0)
    w0 = 1.0 / (1.0 + a)
    e_ref[...] = jnp.concatenate([e0, e1], axis=1)
    w_ref[...] = jnp.concatenate([w0, 1.0 - w0], axis=1)


def _gating(x2d, gate_w, gate_b):
    return pl.pallas_call(
        _gate_body,
        out_shape=(
            jax.ShapeDtypeStruct((S, K), jnp.int32),
            jax.ShapeDtypeStruct((S, K), jnp.float32),
        ),
    )(x2d, gate_w, gate_b.reshape(1, E))


def _lane(vec, e):
    iota16 = lax.iota(jnp.int32, 16)
    return jnp.max(jnp.where(iota16 == e, vec, jnp.int32(-2147483647)))


def _route_body(eflat, wflat, x2d, xs, wsort, invpos, gtile, nact,
                eb, wb, sbuf, shv, posa, posb, toka, tokb,
                rowsa, rowsb, s16, sh, sem, sem2, sem3):
    sid = lax.axis_index("s")
    cid = lax.axis_index("c")
    wid = sid * 2 + cid
    iota16 = lax.iota(jnp.int32, 16)
    blk = N // 16
    pltpu.sync_copy(eflat.at[pl.ds(sid * blk, blk)], eb)
    pltpu.sync_copy(wflat.at[pl.ds(wid * PPW, PPW)], wb)

    loc = jnp.zeros((16,), jnp.int32)
    hlf = jnp.zeros((16,), jnp.int32)
    for cc in range(blk // 16):
        if cc == blk // 32:
            hlf = loc
        ev = eb[pl.ds(cc * 16, 16)]
        upd = jnp.zeros((16,), jnp.int32)
        for e in range(E):
            pc = plsc.all_reduce_population_count(ev == e)
            upd = upd + jnp.where(iota16 == e, pc, 0)
        loc = loc + upd
    sbuf[pl.ds(0, 16)] = loc
    sbuf[pl.ds(16, 16)] = hlf
    pltpu.sync_copy(sbuf, sh.at[pl.ds(sid * 32, 32)])
    plsc.subcore_barrier()
    pltpu.sync_copy(sh, shv)

    cnt = jnp.zeros((16,), jnp.int32)
    pre = jnp.zeros((16,), jnp.int32)
    for s2 in range(16):
        row = shv[pl.ds(s2 * 32, 16)]
        cnt = cnt + row
        pre = pre + jnp.where(jnp.broadcast_to(s2 < sid, (16,)), row, 0)
    own_half = sbuf[pl.ds(16, 16)]
    pre = pre + jnp.where(jnp.broadcast_to(cid == 1, (16,)), own_half, 0)

    padded = ((cnt + (T - 1)) // T) * T
    incl = plsc.cumsum(padded)
    base = (incl - padded) + pre

    half = PPW // 2
    for c2 in range(PPW // 16):
        ev = eb[pl.ds(cid * PPW + c2 * 16, 16)]
        pos = jnp.zeros((16,), jnp.int32)
        delta = jnp.zeros((16,), jnp.int32)
        for e in range(E):
            m = ev == e
            r = plsc.cumsum(jnp.where(m, 1, 0))
            pos = jnp.where(m, _lane(base, e) + r - 1, pos)
            pc = plsc.all_reduce_population_count(m)
            delta = delta + jnp.where(iota16 == e, pc, 0)
        base = base + delta
        p_glob = wid * PPW + c2 * 16 + iota16
        tok = p_glob - jnp.where(p_glob >= S, S, 0)
        hi = c2 >= (PPW // 32)
        dst_pos, dst_tok = (posb, tokb) if hi else (posa, toka)
        off = (c2 - (PPW // 32)) * 16 if hi else c2 * 16
        dst_pos[pl.ds(off, 16)] = pos
        dst_tok[pl.ds(off, 16)] = tok

    ipa = pltpu.async_copy(posa, invpos.at[pl.ds(wid * PPW, half)], sem3)
    ga = pltpu.async_copy(x2d.at[toka], rowsa, sem)
    gb = pltpu.async_copy(x2d.at[tokb], rowsb, sem2)
    ga.wait()
    sa = pltpu.async_copy(rowsa, xs.at[posa], sem)
    gb.wait()
    sb = pltpu.async_copy(rowsb, xs.at[posb], sem2)
    ipa.wait()
    ipb = pltpu.async_copy(posb, invpos.at[pl.ds(wid * PPW + half, half)], sem3)
    sa.wait()
    wa = pltpu.async_copy(wb.at[pl.ds(0, half)], wsort.at[posa], sem)
    sb.wait()
    wb2 = pltpu.async_copy(wb.at[pl.ds(half, half)], wsort.at[posb], sem2)
    ipb.wait()
    wa.wait()
    wb2.wait()

    @pl.when(wid == 0)
    def _():
        la = jnp.max(jnp.where((cnt > 0) & (iota16 < E), iota16, 0))
        total = jnp.max(incl)
        s16[...] = jnp.broadcast_to(total // T, (16,))
        pltpu.sync_copy(s16, nact)
        for j in range(3):
            tstart = (j * 16 + iota16) * T
            acc = jnp.zeros((16,), jnp.int32)
            for e in range(E):
                acc = acc + jnp.where(tstart >= _lane(incl, e), 1, 0)
            s16[...] = jnp.minimum(acc, la)
            pltpu.sync_copy(s16, gtile.at[pl.ds(j * 16, 16)])


def _route(eflat, wflat, x2d):
    return pl.kernel(
        _route_body,
        out_type=(
            jax.ShapeDtypeStruct((P, D), jnp.float32),
            jax.ShapeDtypeStruct((P,), jnp.float32),
            jax.ShapeDtypeStruct((N,), jnp.int32),
            jax.ShapeDtypeStruct((48,), jnp.int32),
            jax.ShapeDtypeStruct((16,), jnp.int32),
        ),
        mesh=_mesh(),
        compiler_params=pltpu.CompilerParams(needs_layout_passes=False),
        scratch_types=[
            pltpu.VMEM((N // 16,), jnp.int32),
            pltpu.VMEM((PPW,), jnp.float32),
            pltpu.VMEM((32,), jnp.int32),
            pltpu.VMEM((512,), jnp.int32),
            pltpu.VMEM((PPW // 2,), jnp.int32),
            pltpu.VMEM((PPW // 2,), jnp.int32),
            pltpu.VMEM((PPW // 2,), jnp.int32),
            pltpu.VMEM((PPW // 2,), jnp.int32),
            pltpu.VMEM((PPW // 2, D), jnp.float32),
            pltpu.VMEM((PPW // 2, D), jnp.float32),
            pltpu.VMEM((16,), jnp.int32),
            pltpu.VMEM_SHARED((512,), jnp.int32),
            pltpu.SemaphoreType.DMA,
            pltpu.SemaphoreType.DMA,
            pltpu.SemaphoreType.DMA,
        ],
    )(eflat, wflat, x2d)


def _gemm_body(g_ref, na_ref, xs_ref, wsc_ref, w1_ref, b1_ref, w2_ref,
               b2_ref, out_ref):
    i = pl.program_id(0)

    @pl.when(i < na_ref[0])
    def _():
        xb = xs_ref[...].astype(jnp.bfloat16)
        w1b = w1_ref[0].astype(jnp.bfloat16)
        h = jnp.dot(xb, w1b, preferred_element_type=jnp.float32)
        h = jnp.maximum(h + b1_ref[0], 0.0).astype(jnp.bfloat16)
        w2b = w2_ref[0].astype(jnp.bfloat16)
        o = jnp.dot(h, w2b, preferred_element_type=jnp.float32)
        out_ref[...] = (o + b2_ref[0]) * wsc_ref[...]


def _grouped_gemm(gtile, nact, xs, wsc, w1, b1, w2, b2):
    grid_spec = pltpu.PrefetchScalarGridSpec(
        num_scalar_prefetch=2,
        grid=(NT,),
        in_specs=[
            pl.BlockSpec((T, D), lambda i, g, na: (i, 0)),
            pl.BlockSpec((T, 1), lambda i, g, na: (i, 0)),
            pl.BlockSpec((1, D, F), lambda i, g, na: (g[i], 0, 0)),
            pl.BlockSpec((1, 1, F), lambda i, g, na: (g[i], 0, 0)),
            pl.BlockSpec((1, F, D), lambda i, g, na: (g[i], 0, 0)),
            pl.BlockSpec((1, 1, D), lambda i, g, na: (g[i], 0, 0)),
        ],
        out_specs=pl.BlockSpec((T, D), lambda i, g, na: (i, 0)),
    )
    return pl.pallas_call(
        _gemm_body,
        grid_spec=grid_spec,
        out_shape=jax.ShapeDtypeStruct((P, D), jnp.float32),
    )(gtile, nact, xs, wsc, w1, b1.reshape(E, 1, F), w2, b2.reshape(E, 1, D))


def _combine_body(contrib, invpos, y, idx, idx2, ra, ra2, sem, sem2):
    wid = _wid()
    base = wid * TPW
    pltpu.sync_copy(invpos.at[pl.ds(base, TPW)], idx)
    ga = pltpu.async_copy(contrib.at[idx], ra, sem)
    pltpu.sync_copy(invpos.at[pl.ds(S + base, TPW)], idx2)
    gb = pltpu.async_copy(contrib.at[idx2], ra2, sem2)
    ga.wait()
    gb.wait()

    def row_body(r, _):
        for u in range(D // 16):
            sl = pl.ds(u * 16, 16)
            ra[r, sl] = ra[r, sl] + ra2[r, sl]
        return 0

    lax.fori_loop(0, TPW, row_body, 0)
    pltpu.sync_copy(ra, y.at[pl.ds(base, TPW)])


def _combine(contrib, invpos):
    return pl.kernel(
        _combine_body,
        out_type=jax.ShapeDtypeStruct((S, D), jnp.float32),
        mesh=_mesh(),
        compiler_params=pltpu.CompilerParams(needs_layout_passes=False),
        scratch_types=[
            pltpu.VMEM((TPW,), jnp.int32),
            pltpu.VMEM((TPW,), jnp.int32),
            pltpu.VMEM((TPW, D), jnp.float32),
            pltpu.VMEM((TPW, D), jnp.float32),
            pltpu.SemaphoreType.DMA,
            pltpu.SemaphoreType.DMA,
        ],
    )(contrib, invpos)


def kernel(x, gate_w, gate_b, w1, b1, w2, b2):
    x2d = x.reshape(S, D)
    e_sk, w_sk = _gating(x2d, gate_w, gate_b)
    eflat = e_sk.T.reshape(N)
    wflat = w_sk.T.reshape(N)
    xs, wsort, invpos, gtile, nact = _route(eflat, wflat, x2d)
    contrib = _grouped_gemm(gtile, nact, xs, wsort.reshape(P, 1),
                            w1, b1, w2, b2)
    y = _combine(contrib, invpos)
    return y.reshape(B, S, D)


# speedup vs baseline: 1.1943x; 1.0156x over previous
import functools

import jax
import jax.numpy as jnp
from jax import lax
from jax.experimental import pallas as pl
from jax.experimental.pallas import tpu as pltpu
from jax.experimental.pallas import tpu_sc as plsc

B, S, D, E, F, K = 1, 2048, 768, 8, 2048, 2
D2 = D // 2
N = S * K
T = 1024
P = N + E * T
NT = P // T
NW = 32
PPW = N // NW
SPW = P // NW
TPW = S // NW

_mesh = functools.partial(
    plsc.VectorSubcoreMesh, core_axis_name="c", subcore_axis_name="s",
    num_cores=2, num_subcores=16)


def _wid():
    return lax.axis_index("s") * 2 + lax.axis_index("c")


def _gate_body(x_ref, gw_ref, gb_ref, e_ref, w_ref):
    x = x_ref[...]
    logits = jnp.dot(x, gw_ref[...], preferred_element_type=jnp.float32)
    logits = logits + gb_ref[...]
    iota = lax.broadcasted_iota(jnp.int32, (S, E), 1)
    m0 = jnp.max(logits, axis=1, keepdims=True)
    e0 = jnp.min(jnp.where(logits == m0, iota, E), axis=1, keepdims=True)
    mask0 = iota == e0
    l1m = jnp.where(mask0, jnp.float32(-1e30), logits)
    m1 = jnp.max(l1m, axis=1, keepdims=True)
    e1 = jnp.min(jnp.where(l1m == m1, iota, E), axis=1, keepdims=True)
    a = jnp.exp(m1 - m0)
    w0 = 1.0 / (1.0 + a)
    e_ref[...] = jnp.concatenate([e0, e1], axis=1)
    w_ref[...] = jnp.concatenate([w0, 1.0 - w0], axis=1)


def _gating(x2d, gate_w, gate_b):
    return pl.pallas_call(
        _gate_body,
        out_shape=(
            jax.ShapeDtypeStruct((S, K), jnp.int32),
            jax.ShapeDtypeStruct((S, K), jnp.float32),
        ),
    )(x2d, gate_w, gate_b.reshape(1, E))


def _lane(vec, e):
    iota16 = lax.iota(jnp.int32, 16)
    return jnp.max(jnp.where(iota16 == e, vec, jnp.int32(-2147483647)))


def _route_body(eflat, wflat, x2d, xs, wsort, invpos, gtile, nact,
                eb, wb, sbuf, shv, posa, posb, toka, tokb,
                rowsa, rowsb, s16, sh, sem, sem2, sem3):
    sid = lax.axis_index("s")
    cid = lax.axis_index("c")
    wid = sid * 2 + cid
    iota16 = lax.iota(jnp.int32, 16)
    blk = N // 16
    pltpu.sync_copy(eflat.at[pl.ds(sid * blk, blk)], eb)
    pltpu.sync_copy(wflat.at[pl.ds(wid * PPW, PPW)], wb)

    loc = jnp.zeros((16,), jnp.int32)
    hlf = jnp.zeros((16,), jnp.int32)
    for cc in range(blk // 16):
        if cc == blk // 32:
            hlf = loc
        ev = eb[pl.ds(cc * 16, 16)]
        upd = jnp.zeros((16,), jnp.int32)
        for e in range(E):
            pc = plsc.all_reduce_population_count(ev == e)
            upd = upd + jnp.where(iota16 == e, pc, 0)
        loc = loc + upd
    sbuf[pl.ds(0, 16)] = loc
    sbuf[pl.ds(16, 16)] = hlf
    pltpu.sync_copy(sbuf, sh.at[pl.ds(sid * 32, 32)])
    plsc.subcore_barrier()
    pltpu.sync_copy(sh, shv)

    cnt = jnp.zeros((16,), jnp.int32)
    pre = jnp.zeros((16,), jnp.int32)
    for s2 in range(16):
        row = shv[pl.ds(s2 * 32, 16)]
        cnt = cnt + row
        pre = pre + jnp.where(jnp.broadcast_to(s2 < sid, (16,)), row, 0)
    own_half = sbuf[pl.ds(16, 16)]
    pre = pre + jnp.where(jnp.broadcast_to(cid == 1, (16,)), own_half, 0)

    padded = ((cnt + (T - 1)) // T) * T
    incl = plsc.cumsum(padded)
    base = (incl - padded) + pre

    half = PPW // 2
    for c2 in range(PPW // 16):
        ev = eb[pl.ds(cid * PPW + c2 * 16, 16)]
        pos = jnp.zeros((16,), jnp.int32)
        delta = jnp.zeros((16,), jnp.int32)
        for e in range(E):
            m = ev == e
            r = plsc.cumsum(jnp.where(m, 1, 0))
            pos = jnp.where(m, _lane(base, e) + r - 1, pos)
            pc = plsc.all_reduce_population_count(m)
            delta = delta + jnp.where(iota16 == e, pc, 0)
        base = base + delta
        p_glob = wid * PPW + c2 * 16 + iota16
        tok = p_glob - jnp.where(p_glob >= S, S, 0)
        hi = c2 >= (PPW // 32)
        dst_pos, dst_tok = (posb, tokb) if hi else (posa, toka)
        off = (c2 - (PPW // 32)) * 16 if hi else c2 * 16
        dst_pos[pl.ds(off, 16)] = pos
        dst_tok[pl.ds(off, 16)] = tok

    ipa = pltpu.async_copy(posa, invpos.at[pl.ds(wid * PPW, half)], sem3)
    ga = pltpu.async_copy(x2d.at[toka], rowsa, sem)
    gb = pltpu.async_copy(x2d.at[tokb], rowsb, sem2)
    ga.wait()
    sa = pltpu.async_copy(rowsa, xs.at[posa], sem)
    gb.wait()
    sb = pltpu.async_copy(rowsb, xs.at[posb], sem2)
    ipa.wait()
    ipb = pltpu.async_copy(posb, invpos.at[pl.ds(wid * PPW + half, half)], sem3)
    sa.wait()
    wa = pltpu.async_copy(wb.at[pl.ds(0, half)], wsort.at[posa], sem)
    sb.wait()
    wb2 = pltpu.async_copy(wb.at[pl.ds(half, half)], wsort.at[posb], sem2)
    ipb.wait()
    wa.wait()
    wb2.wait()

    @pl.when(wid == 0)
    def _():
        la = jnp.max(jnp.where((cnt > 0) & (iota16 < E), iota16, 0))
        total = jnp.max(incl)
        s16[...] = jnp.broadcast_to(total // T, (16,))
        pltpu.sync_copy(s16, nact)
        for j in range(3):
            tstart = (j * 16 + iota16) * T
            acc = jnp.zeros((16,), jnp.int32)
            for e in range(E):
                acc = acc + jnp.where(tstart >= _lane(incl, e), 1, 0)
            s16[...] = jnp.minimum(acc, la)
            pltpu.sync_copy(s16, gtile.at[pl.ds(j * 16, 16)])


def _route(eflat, wflat, x2d):
    return pl.kernel(
        _route_body,
        out_type=(
            jax.ShapeDtypeStruct((P, D), jnp.float32),
            jax.ShapeDtypeStruct((P,), jnp.float32),
            jax.ShapeDtypeStruct((N,), jnp.int32),
            jax.ShapeDtypeStruct((48,), jnp.int32),
            jax.ShapeDtypeStruct((16,), jnp.int32),
        ),
        mesh=_mesh(),
        compiler_params=pltpu.CompilerParams(needs_layout_passes=False),
        scratch_types=[
            pltpu.VMEM((N // 16,), jnp.int32),
            pltpu.VMEM((PPW,), jnp.float32),
            pltpu.VMEM((32,), jnp.int32),
            pltpu.VMEM((512,), jnp.int32),
            pltpu.VMEM((PPW // 2,), jnp.int32),
            pltpu.VMEM((PPW // 2,), jnp.int32),
            pltpu.VMEM((PPW // 2,), jnp.int32),
            pltpu.VMEM((PPW // 2,), jnp.int32),
            pltpu.VMEM((PPW // 2, D), jnp.float32),
            pltpu.VMEM((PPW // 2, D), jnp.float32),
            pltpu.VMEM((16,), jnp.int32),
            pltpu.VMEM_SHARED((512,), jnp.int32),
            pltpu.SemaphoreType.DMA,
            pltpu.SemaphoreType.DMA,
            pltpu.SemaphoreType.DMA,
        ],
    )(eflat, wflat, x2d)


def _gemm_body(g_ref, na_ref, xs_ref, wsc_ref, w1_ref, b1_ref, w2_ref,
               b2_ref, out_ref):
    i = pl.program_id(0)

    @pl.when(i < na_ref[0])
    def _():
        xb = xs_ref[...].astype(jnp.bfloat16)
        w1b = w1_ref[0].astype(jnp.bfloat16)
        h = jnp.dot(xb, w1b, preferred_element_type=jnp.float32)
        h = jnp.maximum(h + b1_ref[0], 0.0).astype(jnp.bfloat16)
        w2b = w2_ref[0].astype(jnp.bfloat16)
        o = jnp.dot(h, w2b, preferred_element_type=jnp.float32)
        out_ref[...] = (o + b2_ref[0]) * wsc_ref[...]


def _grouped_gemm(gtile, nact, xs, wsc, w1, b1, w2, b2):
    grid_spec = pltpu.PrefetchScalarGridSpec(
        num_scalar_prefetch=2,
        grid=(NT,),
        in_specs=[
            pl.BlockSpec((T, D), lambda i, g, na: (i, 0)),
            pl.BlockSpec((T, 1), lambda i, g, na: (i, 0)),
            pl.BlockSpec((1, D, F), lambda i, g, na: (g[i], 0, 0)),
            pl.BlockSpec((1, 1, F), lambda i, g, na: (g[i], 0, 0)),
            pl.BlockSpec((1, F, D), lambda i, g, na: (g[i], 0, 0)),
            pl.BlockSpec((1, 1, D), lambda i, g, na: (g[i], 0, 0)),
        ],
        out_specs=pl.BlockSpec((T, D), lambda i, g, na: (i, 0)),
    )
    return pl.pallas_call(
        _gemm_body,
        grid_spec=grid_spec,
        out_shape=jax.ShapeDtypeStruct((P, D), jnp.float32),
    )(gtile, nact, xs, wsc, w1, b1.reshape(E, 1, F), w2, b2.reshape(E, 1, D))


def _combine_body(contrib, invpos, y, idx, idx2, ra, ra2, sem, sem2):
    wid = _wid()
    base = wid * TPW
    pltpu.sync_copy(invpos.at[pl.ds(base, TPW)], idx)
    ga = pltpu.async_copy(contrib.at[idx], ra, sem)
    pltpu.sync_copy(invpos.at[pl.ds(S + base, TPW)], idx2)
    gb = pltpu.async_copy(contrib.at[idx2], ra2, sem2)
    ga.wait()
    gb.wait()

    def row_body(r, _):
        for u in range(D // 16):
            sl = pl.ds(u * 16, 16)
            ra[r, sl] = ra[r, sl] + ra2[r, sl]
        return 0

    lax.fori_loop(0, TPW, row_body, 0)
    pltpu.sync_copy(ra, y.at[pl.ds(base, TPW)])


def _combine(contrib, invpos):
    return pl.kernel(
        _combine_body,
        out_type=jax.ShapeDtypeStruct((S, D), jnp.float32),
        mesh=_mesh(),
        compiler_params=pltpu.CompilerParams(needs_layout_passes=False),
        scratch_types=[
            pltpu.VMEM((TPW,), jnp.int32),
            pltpu.VMEM((TPW,), jnp.int32),
            pltpu.VMEM((TPW, D), jnp.float32),
            pltpu.VMEM((TPW, D), jnp.float32),
            pltpu.SemaphoreType.DMA,
            pltpu.SemaphoreType.DMA,
        ],
    )(contrib, invpos)


def kernel(x, gate_w, gate_b, w1, b1, w2, b2):
    x2d = x.reshape(S, D)
    e_sk, w_sk = _gating(x2d, gate_w, gate_b)
    eflat = e_sk.T.reshape(N)
    wflat = w_sk.T.reshape(N)
    xs, wsort, invpos, gtile, nact = _route(eflat, wflat, x2d)
    contrib = _grouped_gemm(gtile, nact, xs, wsort.reshape(P, 1),
                            w1, b1, w2, b2)
    y = _combine(contrib, invpos)
    return y.reshape(B, S, D)


# speedup vs baseline: 1.1946x; 1.0003x over previous
import functools

import jax
import jax.numpy as jnp
from jax import lax
from jax.experimental import pallas as pl
from jax.experimental.pallas import tpu as pltpu
from jax.experimental.pallas import tpu_sc as plsc

B, S, D, E, F, K = 1, 2048, 768, 8, 2048, 2
N = S * K
T = 1024
P = N + E * T
NT = P // T
NW = 32
PPW = N // NW
TPW = S // NW

_mesh = functools.partial(
    plsc.VectorSubcoreMesh, core_axis_name="c", subcore_axis_name="s",
    num_cores=2, num_subcores=16)


def _wid():
    return lax.axis_index("s") * 2 + lax.axis_index("c")


def _gate_body(x_ref, gw_ref, gb_ref, e_ref, w_ref):
    x = x_ref[...]
    logits = jnp.dot(x, gw_ref[...], preferred_element_type=jnp.float32)
    logits = logits + gb_ref[...]
    iota = lax.broadcasted_iota(jnp.int32, (S, E), 1)
    m0 = jnp.max(logits, axis=1, keepdims=True)
    e0 = jnp.min(jnp.where(logits == m0, iota, E), axis=1, keepdims=True)
    mask0 = iota == e0
    l1m = jnp.where(mask0, jnp.float32(-1e30), logits)
    m1 = jnp.max(l1m, axis=1, keepdims=True)
    e1 = jnp.min(jnp.where(l1m == m1, iota, E), axis=1, keepdims=True)
    a = jnp.exp(m1 - m0)
    w0 = 1.0 / (1.0 + a)
    e_ref[...] = jnp.concatenate([e0, e1], axis=1)
    w_ref[...] = jnp.concatenate([w0, 1.0 - w0], axis=1)


def _gating(x2d, gate_w, gate_b):
    return pl.pallas_call(
        _gate_body,
        out_shape=(
            jax.ShapeDtypeStruct((S, K), jnp.int32),
            jax.ShapeDtypeStruct((S, K), jnp.float32),
        ),
    )(x2d, gate_w, gate_b.reshape(1, E))


def _lane(vec, e):
    iota16 = lax.iota(jnp.int32, 16)
    return jnp.max(jnp.where(iota16 == e, vec, jnp.int32(-2147483647)))


def _route_body(eflat, wflat, x2d, xs, wsort, invpos, gtile, nact,
                eb, wb, sbuf, shv, posa, posb, toka, tokb,
                rowsa, rowsb, s16, sh, sem, sem2, sem3):
    sid = lax.axis_index("s")
    cid = lax.axis_index("c")
    wid = sid * 2 + cid
    iota16 = lax.iota(jnp.int32, 16)
    blk = N // 16
    pltpu.sync_copy(eflat.at[pl.ds(sid * blk, blk)], eb)
    pltpu.sync_copy(wflat.at[pl.ds(wid * PPW, PPW)], wb)

    loc = jnp.zeros((16,), jnp.int32)
    hlf = jnp.zeros((16,), jnp.int32)
    for cc in range(blk // 16):
        if cc == blk // 32:
            hlf = loc
        ev = eb[pl.ds(cc * 16, 16)]
        upd = jnp.zeros((16,), jnp.int32)
        for e in range(E):
            pc = plsc.all_reduce_population_count(ev == e)
            upd = upd + jnp.where(iota16 == e, pc, 0)
        loc = loc + upd
    sbuf[pl.ds(0, 16)] = loc
    sbuf[pl.ds(16, 16)] = hlf
    pltpu.sync_copy(sbuf, sh.at[pl.ds(sid * 32, 32)])
    plsc.subcore_barrier()
    pltpu.sync_copy(sh, shv)

    cnt = jnp.zeros((16,), jnp.int32)
    pre = jnp.zeros((16,), jnp.int32)
    for s2 in range(16):
        row = shv[pl.ds(s2 * 32, 16)]
        cnt = cnt + row
        pre = pre + jnp.where(jnp.broadcast_to(s2 < sid, (16,)), row, 0)
    own_half = sbuf[pl.ds(16, 16)]
    pre = pre + jnp.where(jnp.broadcast_to(cid == 1, (16,)), own_half, 0)

    padded = ((cnt + (T - 1)) // T) * T
    incl = plsc.cumsum(padded)
    base = (incl - padded) + pre

    half = PPW // 2
    for c2 in range(PPW // 16):
        ev = eb[pl.ds(cid * PPW + c2 * 16, 16)]
        pos = jnp.zeros((16,), jnp.int32)
        delta = jnp.zeros((16,), jnp.int32)
        for e in range(E):
            m = ev == e
            r = plsc.cumsum(jnp.where(m, 1, 0))
            pos = jnp.where(m, _lane(base, e) + r - 1, pos)
            pc = plsc.all_reduce_population_count(m)
            delta = delta + jnp.where(iota16 == e, pc, 0)
        base = base + delta
        p_glob = wid * PPW + c2 * 16 + iota16
        tok = p_glob - jnp.where(p_glob >= S, S, 0)
        hi = c2 >= (PPW // 32)
        dst_pos, dst_tok = (posb, tokb) if hi else (posa, toka)
        off = (c2 - (PPW // 32)) * 16 if hi else c2 * 16
        dst_pos[pl.ds(off, 16)] = pos
        dst_tok[pl.ds(off, 16)] = tok

    ipa = pltpu.async_copy(posa, invpos.at[pl.ds(wid * PPW, half)], sem3)
    ga = pltpu.async_copy(x2d.at[toka], rowsa, sem)
    gb = pltpu.async_copy(x2d.at[tokb], rowsb, sem2)
    ga.wait()
    sa = pltpu.async_copy(rowsa, xs.at[posa], sem)
    gb.wait()
    sb = pltpu.async_copy(rowsb, xs.at[posb], sem2)
    ipa.wait()
    ipb = pltpu.async_copy(posb, invpos.at[pl.ds(wid * PPW + half, half)], sem3)
    sa.wait()
    wa = pltpu.async_copy(wb.at[pl.ds(0, half)], wsort.at[posa], sem)
    sb.wait()
    wb2 = pltpu.async_copy(wb.at[pl.ds(half, half)], wsort.at[posb], sem2)
    ipb.wait()
    wa.wait()
    wb2.wait()

    @pl.when(wid == 0)
    def _():
        la = jnp.max(jnp.where((cnt > 0) & (iota16 < E), iota16, 0))
        total = jnp.max(incl)
        s16[...] = jnp.broadcast_to(total // T, (16,))
        pltpu.sync_copy(s16, nact)
        for j in range(3):
            tstart = (j * 16 + iota16) * T
            acc = jnp.zeros((16,), jnp.int32)
            for e in range(E):
                acc = acc + jnp.where(tstart >= _lane(incl, e), 1, 0)
            s16[...] = jnp.minimum(acc, la)
            pltpu.sync_copy(s16, gtile.at[pl.ds(j * 16, 16)])


def _route(eflat, wflat, x2d):
    return pl.kernel(
        _route_body,
        out_type=(
            jax.ShapeDtypeStruct((P, D), jnp.float32),
            jax.ShapeDtypeStruct((P,), jnp.float32),
            jax.ShapeDtypeStruct((N,), jnp.int32),
            jax.ShapeDtypeStruct((48,), jnp.int32),
            jax.ShapeDtypeStruct((16,), jnp.int32),
        ),
        mesh=_mesh(),
        compiler_params=pltpu.CompilerParams(needs_layout_passes=False),
        scratch_types=[
            pltpu.VMEM((N // 16,), jnp.int32),
            pltpu.VMEM((PPW,), jnp.float32),
            pltpu.VMEM((32,), jnp.int32),
            pltpu.VMEM((512,), jnp.int32),
            pltpu.VMEM((PPW // 2,), jnp.int32),
            pltpu.VMEM((PPW // 2,), jnp.int32),
            pltpu.VMEM((PPW // 2,), jnp.int32),
            pltpu.VMEM((PPW // 2,), jnp.int32),
            pltpu.VMEM((PPW // 2, D), jnp.float32),
            pltpu.VMEM((PPW // 2, D), jnp.float32),
            pltpu.VMEM((16,), jnp.int32),
            pltpu.VMEM_SHARED((512,), jnp.int32),
            pltpu.SemaphoreType.DMA,
            pltpu.SemaphoreType.DMA,
            pltpu.SemaphoreType.DMA,
        ],
    )(eflat, wflat, x2d)


def _gemm_body(g_ref, na_ref, xs_ref, wsc_ref, w1_ref, b1_ref, w2_ref,
               b2_ref, out_ref):
    i = pl.program_id(0)

    @pl.when(i < na_ref[0])
    def _():
        xb = xs_ref[...].astype(jnp.bfloat16)
        w1b = w1_ref[0].astype(jnp.bfloat16)
        h = jnp.dot(xb, w1b, preferred_element_type=jnp.float32)
        h = jnp.maximum(h + b1_ref[0], 0.0).astype(jnp.bfloat16)
        w2b = w2_ref[0].astype(jnp.bfloat16)
        o = jnp.dot(h, w2b, preferred_element_type=jnp.float32)
        out_ref[...] = (o + b2_ref[0]) * wsc_ref[...]


def _grouped_gemm(gtile, nact, xs, wsc, w1, b1, w2, b2):
    grid_spec = pltpu.PrefetchScalarGridSpec(
        num_scalar_prefetch=2,
        grid=(NT,),
        in_specs=[
            pl.BlockSpec((T, D), lambda i, g, na: (i, 0)),
            pl.BlockSpec((T, 1), lambda i, g, na: (i, 0)),
            pl.BlockSpec((1, D, F), lambda i, g, na: (g[i], 0, 0)),
            pl.BlockSpec((1, 1, F), lambda i, g, na: (g[i], 0, 0)),
            pl.BlockSpec((1, F, D), lambda i, g, na: (g[i], 0, 0)),
            pl.BlockSpec((1, 1, D), lambda i, g, na: (g[i], 0, 0)),
        ],
        out_specs=pl.BlockSpec((T, D), lambda i, g, na: (i, 0)),
    )
    return pl.pallas_call(
        _gemm_body,
        grid_spec=grid_spec,
        out_shape=jax.ShapeDtypeStruct((P, D), jnp.float32),
    )(gtile, nact, xs, wsc, w1, b1.reshape(E, 1, F), w2, b2.reshape(E, 1, D))


def _combine_body(contrib, invpos, y, idx, idx2, ra, ra2, sem, sem2):
    wid = _wid()
    base = wid * TPW
    pltpu.sync_copy(invpos.at[pl.ds(base, TPW)], idx)
    ga = pltpu.async_copy(contrib.at[idx], ra, sem)
    pltpu.sync_copy(invpos.at[pl.ds(S + base, TPW)], idx2)
    gb = pltpu.async_copy(contrib.at[idx2], ra2, sem2)
    ga.wait()
    gb.wait()

    def row_body(r, _):
        for u in range(D // 16):
            sl = pl.ds(u * 16, 16)
            ra[r, sl] = ra[r, sl] + ra2[r, sl]
        return 0

    lax.fori_loop(0, TPW, row_body, 0)
    pltpu.sync_copy(ra, y.at[pl.ds(base, TPW)])


def _combine(contrib, invpos):
    return pl.kernel(
        _combine_body,
        out_type=jax.ShapeDtypeStruct((S, D), jnp.float32),
        mesh=_mesh(),
        compiler_params=pltpu.CompilerParams(needs_layout_passes=False),
        scratch_types=[
            pltpu.VMEM((TPW,), jnp.int32),
            pltpu.VMEM((TPW,), jnp.int32),
            pltpu.VMEM((TPW, D), jnp.float32),
            pltpu.VMEM((TPW, D), jnp.float32),
            pltpu.SemaphoreType.DMA,
            pltpu.SemaphoreType.DMA,
        ],
    )(contrib, invpos)


def kernel(x, gate_w, gate_b, w1, b1, w2, b2):
    x2d = x.reshape(S, D)
    e_sk, w_sk = _gating(x2d, gate_w, gate_b)
    eflat = e_sk.T.reshape(N)
    wflat = w_sk.T.reshape(N)
    xs, wsort, invpos, gtile, nact = _route(eflat, wflat, x2d)
    contrib = _grouped_gemm(gtile, nact, xs, wsort.reshape(P, 1),
                            w1, b1, w2, b2)
    y = _combine(contrib, invpos)
    return y.reshape(B, S, D)
